# Initial kernel scaffold; baseline (speedup 1.0000x reference)
import jax
import jax.numpy as jnp
from jax.experimental import pallas as pl


def kernel(node_features, edge_features, edge_index, params):
    raise NotImplementedError("write your pallas kernel here")


# speedup vs baseline: 1.3578x; 1.3578x over previous
import dataclasses
import functools

import jax
import jax.numpy as jnp
from jax import lax
from jax.experimental import pallas as pl
from jax.experimental.pallas import tpu as pltpu
from jax.experimental.pallas import tpu_sc as plsc

F32 = jnp.float32


def _sc_compiler_params():
    cp = pltpu.CompilerParams()
    if "needs_layout_passes" in pltpu.CompilerParams.__dataclass_fields__:
        cp = dataclasses.replace(cp, needs_layout_passes=False)
    return cp


def _gelu(x):
    return x * 0.5 * (1.0 + lax.erf(x * (2.0 ** -0.5)))


def _ln_rows(r, g, b, eps=1e-5):
    m = jnp.mean(r, axis=1, keepdims=True)
    c = r - m
    v = jnp.mean(c * c, axis=1, keepdims=True)
    return c * jax.lax.rsqrt(v + eps) * g + b


def _proj_body(nf, w_all, b_all, ts, td, pn):
    x = nf[...]
    h1 = jnp.dot(x, w_all[...], preferred_element_type=F32) + b_all[...]
    ts[...] = h1[:, 0:512]
    td[...] = h1[:, 512:1024]
    pn[...] = h1[:, 1024:1280]


def _node_proj(nf, w_all, b_all, blk):
    n, h = nf.shape
    grid = n // blk
    return pl.pallas_call(
        _proj_body,
        grid=(grid,),
        in_specs=[pl.BlockSpec((blk, h), lambda i: (i, 0)),
                  pl.BlockSpec(w_all.shape, lambda i: (0, 0)),
                  pl.BlockSpec(b_all.shape, lambda i: (0, 0))],
        out_specs=[pl.BlockSpec((blk, 2 * h), lambda i: (i, 0)),
                   pl.BlockSpec((blk, 2 * h), lambda i: (i, 0)),
                   pl.BlockSpec((blk, h), lambda i: (i, 0))],
        out_shape=[jax.ShapeDtypeStruct((n, 2 * h), F32),
                   jax.ShapeDtypeStruct((n, 2 * h), F32),
                   jax.ShapeDtypeStruct((n, h), F32)],
    )(nf, w_all, b_all)


def _edge_gather(tsrc, tdst, src1d, dst1d):
    e = src1d.shape[0]
    d = tsrc.shape[1]
    nw = 32
    eps = e // nw
    gw = 40
    mesh = plsc.VectorSubcoreMesh(core_axis_name="core", subcore_axis_name="subcore")

    @functools.partial(
        pl.kernel,
        out_type=[jax.ShapeDtypeStruct((e, d), F32),
                  jax.ShapeDtypeStruct((e, d), F32)],
        mesh=mesh,
        scratch_types=[pltpu.VMEM((gw,), jnp.int32),
                       pltpu.VMEM((gw,), jnp.int32),
                       pltpu.VMEM((gw, d), F32),
                       pltpu.VMEM((gw, d), F32),
                       pltpu.SemaphoreType.DMA,
                       pltpu.SemaphoreType.DMA],
        compiler_params=_sc_compiler_params(),
    )
    def gather_kernel(tsrc_hbm, tdst_hbm, src_hbm, dst_hbm, osrc_hbm, odst_hbm,
                      idx1, idx2, rows1, rows2, sem1, sem2):
        t = lax.axis_index("subcore") * 2 + lax.axis_index("core")
        base0 = t * eps

        @pl.loop(0, eps, step=gw)
        def _(i):
            b = base0 + i
            pltpu.sync_copy(src_hbm.at[pl.ds(b, gw)], idx1)
            pltpu.sync_copy(dst_hbm.at[pl.ds(b, gw)], idx2)
            c1 = pltpu.async_copy(tsrc_hbm.at[idx1], rows1, sem1)
            c2 = pltpu.async_copy(tdst_hbm.at[idx2], rows2, sem2)
            c1.wait()
            c2.wait()
            pltpu.sync_copy(rows1, osrc_hbm.at[pl.ds(b, gw)])
            pltpu.sync_copy(rows2, odst_hbm.at[pl.ds(b, gw)])

    return gather_kernel(tsrc, tdst, src1d, dst1d)


def _edge_body(gs, gd, ef, wqm, wqe, mw2, ew2, eb2, lg, lb, msgt, ne):
    x = ef[...]
    gsv = gs[...]
    gdv = gd[...]
    pre_m = (gsv[:, 0:256] + gdv[:, 0:256]
             + jnp.dot(x, wqm[...], preferred_element_type=F32))
    h_m = _gelu(pre_m)
    msgt[...] = lax.dot_general(mw2[...], h_m,
                                dimension_numbers=(((0,), (1,)), ((), ())),
                                preferred_element_type=F32)
    pre_e = (gsv[:, 256:512] + gdv[:, 256:512]
             + jnp.dot(x, wqe[...], preferred_element_type=F32))
    h_e = _gelu(pre_e)
    eu = jnp.dot(h_e, ew2[...], preferred_element_type=F32) + eb2[...]
    ne[...] = _ln_rows(x + eu, lg[...], lb[...])


def _edge_mlp(gs, gd, ef, wqm, wqe, mw2, ew2, eb2, lg, lb, blk):
    e, h = ef.shape
    grid = e // blk
    full = lambda a: pl.BlockSpec(a.shape, lambda i: (0, 0))
    rowblk = pl.BlockSpec((blk, h), lambda i: (i, 0))
    wideblk = pl.BlockSpec((blk, 2 * h), lambda i: (i, 0))
    return pl.pallas_call(
        _edge_body,
        grid=(grid,),
        in_specs=[wideblk, wideblk, rowblk,
                  full(wqm), full(wqe), full(mw2), full(ew2),
                  full(eb2), full(lg), full(lb)],
        out_specs=[pl.BlockSpec((h, blk), lambda i: (0, i)), rowblk],
        out_shape=[jax.ShapeDtypeStruct((h, e), F32),
                   jax.ShapeDtypeStruct((e, h), F32)],
    )(gs, gd, ef, wqm, wqe, mw2, ew2, eb2, lg, lb)


def _scatter_agg(msgt, dst1d, n_nodes):
    h, e = msgt.shape
    ngroups = 32
    gw = h // ngroups
    npad = ((n_nodes + 7) // 8) * 8 + 48
    acc_len = npad * gw
    dlen = ((n_nodes + 15) // 16) * 16
    ch = 640
    nch = e // ch
    eps = e // 32
    mesh = plsc.VectorSubcoreMesh(core_axis_name="core", subcore_axis_name="subcore")

    @functools.partial(
        pl.kernel,
        out_type=[jax.ShapeDtypeStruct((32 * acc_len,), F32),
                  jax.ShapeDtypeStruct((32 * dlen,), F32)],
        mesh=mesh,
        scratch_types=[pltpu.VMEM((acc_len,), F32),
                       pltpu.VMEM((dlen,), F32),
                       pltpu.VMEM((gw, ch), F32),
                       pltpu.VMEM((gw, ch), F32),
                       pltpu.VMEM((ch,), jnp.int32),
                       pltpu.VMEM((ch,), jnp.int32),
                       pltpu.VMEM((ch,), jnp.int32),
                       pltpu.VMEM((eps,), jnp.int32),
                       pltpu.SemaphoreType.DMA,
                       pltpu.SemaphoreType.DMA,
                       pltpu.SemaphoreType.DMA,
                       pltpu.SemaphoreType.DMA],
        compiler_params=_sc_compiler_params(),
    )
    def scatter_kernel(msgt_hbm, dst_hbm, agg_hbm, deg_hbm,
                       acc1, dhist, vals0, vals1,
                       raw0, raw1, raw8, rawd, semg0, semg1, semr0, semr1):
        t = lax.axis_index("subcore") * 2 + lax.axis_index("core")
        iota = lax.iota(jnp.int32, 16)
        zero16 = jnp.zeros((16,), F32)
        ones16 = jnp.ones((16,), F32)

        valsb = (vals0, vals1)
        rawb = (raw0, raw1)
        semgb = (semg0, semg1)
        semrb = (semr0, semr1)

        @pl.loop(0, acc_len, step=16)
        def _(i):
            acc1[pl.ds(i, 16)] = zero16

        @pl.loop(0, dlen, step=16)
        def _(i):
            dhist[pl.ds(i, 16)] = zero16

        def issue(mm, b):
            e0 = mm * ch
            pltpu.async_copy(msgt_hbm.at[pl.ds(t * gw, gw), pl.ds(e0, ch)],
                             valsb[b], semgb[b])
            pltpu.async_copy(dst_hbm.at[pl.ds(e0, ch)], rawb[b], semrb[b])

        def wait(mm, b):
            e0 = mm * ch
            pltpu.make_async_copy(msgt_hbm.at[pl.ds(t * gw, gw), pl.ds(e0, ch)],
                                  valsb[b], semgb[b]).wait()
            pltpu.make_async_copy(dst_hbm.at[pl.ds(e0, ch)], rawb[b], semrb[b]).wait()

        def compute(b):
            @pl.loop(0, ch, step=16)
            def _(k):
                raw8[pl.ds(k, 16)] = rawb[b][pl.ds(k, 16)] * gw

            @pl.loop(0, ch, step=16)
            def _(j):
                offs0 = raw8[pl.ds(j, 16)]
                for r in range(gw):
                    vv = valsb[b][r, pl.ds(j, 16)]
                    plsc.addupdate_scatter(acc1, [offs0 + r], vv)

        issue(0, 0)

        @pl.loop(0, nch, step=2)
        def _(m):
            for db in range(2):
                mm = m + db

                @pl.when(mm + 1 < nch)
                def _():
                    issue(mm + 1, 1 - db)

                wait(mm, db)
                compute(db)

        e0d = t * eps
        pltpu.sync_copy(dst_hbm.at[pl.ds(e0d, eps)], rawd)

        full16 = (eps // 16) * 16

        @pl.loop(0, full16, step=16)
        def _(j):
            plsc.addupdate_scatter(dhist, [rawd[pl.ds(j, 16)]], ones16)

        if eps != full16:
            tail = rawd[pl.ds(eps - 16, 16)]
            plsc.addupdate_scatter(dhist, [tail], ones16,
                                   mask=iota >= (16 - (eps - full16)))

        pltpu.sync_copy(acc1, agg_hbm.at[pl.ds(t * acc_len, acc_len)])
        pltpu.sync_copy(dhist, deg_hbm.at[pl.ds(t * dlen, dlen)])

    agg_f, deg_f = scatter_kernel(msgt, dst1d)
    agg = agg_f.reshape(32, npad, gw).transpose(1, 0, 2).reshape(npad, h)[:n_nodes]
    deg_t = deg_f.reshape(32, dlen)[:, :n_nodes].T
    return agg, deg_t


def _node_body(nf, pn, agg, deg, w1d, nb1, mb2r, nw2, nb2, lg, lb, out):
    degree = jnp.sum(deg[...], axis=1, keepdims=True)
    aggn = (agg[...] + degree * mb2r[...]) / (degree + 1e-8)
    pre = pn[...] + jnp.dot(aggn, w1d[...], preferred_element_type=F32) + nb1[...]
    hh = _gelu(pre)
    upd = jnp.dot(hh, nw2[...], preferred_element_type=F32) + nb2[...]
    out[...] = _ln_rows(nf[...] + upd, lg[...], lb[...])


def _node_update(nf, pn, agg, deg, w1d, nb1, mb2r, nw2, nb2, lg, lb, blk):
    n, h = nf.shape
    grid = n // blk
    full = lambda a: pl.BlockSpec(a.shape, lambda i: (0, 0))
    rowblk = pl.BlockSpec((blk, h), lambda i: (i, 0))
    return pl.pallas_call(
        _node_body,
        grid=(grid,),
        in_specs=[rowblk, rowblk, rowblk,
                  pl.BlockSpec((blk, 32), lambda i: (i, 0)),
                  full(w1d), full(nb1), full(mb2r), full(nw2), full(nb2),
                  full(lg), full(lb)],
        out_specs=rowblk,
        out_shape=jax.ShapeDtypeStruct((n, h), F32),
    )(nf, pn, agg, deg, w1d, nb1, mb2r, nw2, nb2, lg, lb)


def kernel(node_features, edge_features, edge_index, params):
    p = params
    n, h = node_features.shape
    e = edge_features.shape[0]

    w_all = jnp.concatenate(
        [p['mW1'][0:h], p['eW1'][0:h],
         p['mW1'][h:2 * h], p['eW1'][h:2 * h],
         p['nW1'][0:h]], axis=1)
    b_all = jnp.concatenate(
        [p['mb1'], p['eb1'], jnp.zeros((3 * h,), F32)]).reshape(1, 5 * h)

    tsrc, tdst, pn = _node_proj(node_features, w_all, b_all, blk=1000)

    src = edge_index[0]
    dst = edge_index[1]
    gsrc, gdst = _edge_gather(tsrc, tdst, src, dst)

    msgt, new_edge = _edge_mlp(
        gsrc, gdst, edge_features,
        p['mW1'][2 * h:3 * h], p['eW1'][2 * h:3 * h],
        p['mW2'], p['eW2'], p['eb2'].reshape(1, h),
        p['edge_ln_g'].reshape(1, h), p['edge_ln_b'].reshape(1, h), blk=1280)

    agg, deg = _scatter_agg(msgt, dst, n)

    new_node = _node_update(
        node_features, pn, agg, deg,
        p['nW1'][h:2 * h], p['nb1'].reshape(1, h), p['mb2'].reshape(1, h),
        p['nW2'], p['nb2'].reshape(1, h),
        p['node_ln_g'].reshape(1, h), p['node_ln_b'].reshape(1, h), blk=1000)

    return (new_node, new_edge)


# speedup vs baseline: 1.5031x; 1.1070x over previous
import dataclasses
import functools

import jax
import jax.numpy as jnp
from jax import lax
from jax.experimental import pallas as pl
from jax.experimental.pallas import tpu as pltpu
from jax.experimental.pallas import tpu_sc as plsc

F32 = jnp.float32


def _sc_compiler_params():
    cp = pltpu.CompilerParams()
    if "needs_layout_passes" in pltpu.CompilerParams.__dataclass_fields__:
        cp = dataclasses.replace(cp, needs_layout_passes=False)
    return cp


def _gelu(x):
    return x * 0.5 * (1.0 + lax.erf(x * (2.0 ** -0.5)))


def _ln_rows(r, g, b, eps=1e-5):
    m = jnp.mean(r, axis=1, keepdims=True)
    c = r - m
    v = jnp.mean(c * c, axis=1, keepdims=True)
    return c * jax.lax.rsqrt(v + eps) * g + b


def _proj_body(nf, w_all, b_all, ts, td, pn):
    x = nf[...]
    h1 = jnp.dot(x, w_all[...], preferred_element_type=F32) + b_all[...]
    ts[...] = h1[:, 0:512]
    td[...] = h1[:, 512:1024]
    pn[...] = h1[:, 1024:1280]


def _node_proj(nf, w_all, b_all, blk):
    n, h = nf.shape
    grid = n // blk
    return pl.pallas_call(
        _proj_body,
        grid=(grid,),
        in_specs=[pl.BlockSpec((blk, h), lambda i: (i, 0)),
                  pl.BlockSpec(w_all.shape, lambda i: (0, 0)),
                  pl.BlockSpec(b_all.shape, lambda i: (0, 0))],
        out_specs=[pl.BlockSpec((blk, 2 * h), lambda i: (i, 0)),
                   pl.BlockSpec((blk, 2 * h), lambda i: (i, 0)),
                   pl.BlockSpec((blk, h), lambda i: (i, 0))],
        out_shape=[jax.ShapeDtypeStruct((n, 2 * h), F32),
                   jax.ShapeDtypeStruct((n, 2 * h), F32),
                   jax.ShapeDtypeStruct((n, h), F32)],
    )(nf, w_all, b_all)


def _edge_gather(tsrc, tdst, src1d, dst1d):
    e = src1d.shape[0]
    d = tsrc.shape[1]
    nw = 32
    eps = e // nw
    gw = 40
    mesh = plsc.VectorSubcoreMesh(core_axis_name="core", subcore_axis_name="subcore")

    @functools.partial(
        pl.kernel,
        out_type=[jax.ShapeDtypeStruct((e, d), F32),
                  jax.ShapeDtypeStruct((e, d), F32)],
        mesh=mesh,
        scratch_types=[pltpu.VMEM((2, gw), jnp.int32),
                       pltpu.VMEM((2, gw), jnp.int32),
                       pltpu.VMEM((2, gw, d), F32),
                       pltpu.VMEM((2, gw, d), F32),
                       pltpu.SemaphoreType.DMA,
                       pltpu.SemaphoreType.DMA,
                       pltpu.SemaphoreType.DMA,
                       pltpu.SemaphoreType.DMA],
        compiler_params=_sc_compiler_params(),
    )
    def gather_kernel(tsrc_hbm, tdst_hbm, src_hbm, dst_hbm, osrc_hbm, odst_hbm,
                      idx1, idx2, rows1, rows2, sem1a, sem1b_, sem2a, sem2b_):
        t = lax.axis_index("subcore") * 2 + lax.axis_index("core")
        base0 = t * eps
        nsteps = eps // gw

        idx1b = (idx1.at[0], idx1.at[1])
        idx2b = (idx2.at[0], idx2.at[1])
        rows1b = (rows1.at[0], rows1.at[1])
        rows2b = (rows2.at[0], rows2.at[1])
        sem1b = (sem1a, sem1b_)
        sem2b = (sem2a, sem2b_)

        def start(s, db):
            b = base0 + s * gw
            pltpu.sync_copy(src_hbm.at[pl.ds(b, gw)], idx1b[db])
            pltpu.sync_copy(dst_hbm.at[pl.ds(b, gw)], idx2b[db])
            pltpu.async_copy(tsrc_hbm.at[idx1b[db]], rows1b[db], sem1b[db])
            pltpu.async_copy(tdst_hbm.at[idx2b[db]], rows2b[db], sem2b[db])

        def finish(s, db):
            b = base0 + s * gw
            pltpu.make_async_copy(tsrc_hbm.at[idx1b[db]], rows1b[db],
                                  sem1b[db]).wait()
            pltpu.make_async_copy(tdst_hbm.at[idx2b[db]], rows2b[db],
                                  sem2b[db]).wait()
            pltpu.sync_copy(rows1b[db], osrc_hbm.at[pl.ds(b, gw)])
            pltpu.sync_copy(rows2b[db], odst_hbm.at[pl.ds(b, gw)])

        start(0, 0)

        @pl.loop(0, nsteps - 1, step=2)
        def _(m):
            for db in range(2):
                mm = m + db

                @pl.when(mm + 1 < nsteps)
                def _():
                    start(mm + 1, 1 - db)

                finish(mm, db)

        finish(nsteps - 1, (nsteps - 1) % 2)

    return gather_kernel(tsrc, tdst, src1d, dst1d)


def _edge_body(gs, gd, ef, wqm, wqe, mw2, ew2, eb2, lg, lb, msgt, ne):
    x = ef[...]
    gsv = gs[...]
    gdv = gd[...]
    pre_m = (gsv[:, 0:256] + gdv[:, 0:256]
             + jnp.dot(x, wqm[...], preferred_element_type=F32))
    h_m = _gelu(pre_m)
    msgt[...] = lax.dot_general(mw2[...], h_m,
                                dimension_numbers=(((0,), (1,)), ((), ())),
                                preferred_element_type=F32)
    pre_e = (gsv[:, 256:512] + gdv[:, 256:512]
             + jnp.dot(x, wqe[...], preferred_element_type=F32))
    h_e = _gelu(pre_e)
    eu = jnp.dot(h_e, ew2[...], preferred_element_type=F32) + eb2[...]
    ne[...] = _ln_rows(x + eu, lg[...], lb[...])


def _edge_mlp(gs, gd, ef, wqm, wqe, mw2, ew2, eb2, lg, lb, blk):
    e, h = ef.shape
    grid = e // blk
    full = lambda a: pl.BlockSpec(a.shape, lambda i: (0, 0))
    rowblk = pl.BlockSpec((blk, h), lambda i: (i, 0))
    wideblk = pl.BlockSpec((blk, 2 * h), lambda i: (i, 0))
    return pl.pallas_call(
        _edge_body,
        grid=(grid,),
        in_specs=[wideblk, wideblk, rowblk,
                  full(wqm), full(wqe), full(mw2), full(ew2),
                  full(eb2), full(lg), full(lb)],
        out_specs=[pl.BlockSpec((h, blk), lambda i: (0, i)), rowblk],
        out_shape=[jax.ShapeDtypeStruct((h, e), F32),
                   jax.ShapeDtypeStruct((e, h), F32)],
    )(gs, gd, ef, wqm, wqe, mw2, ew2, eb2, lg, lb)


def _scatter_agg(msgt, dst1d, n_nodes):
    h, e = msgt.shape
    ngroups = 32
    gw = h // ngroups
    npad = ((n_nodes + 7) // 8) * 8 + 48
    acc_len = npad * gw
    dlen = ((n_nodes + 15) // 16) * 16
    ch = 640
    nch = e // ch
    eps = e // 32
    mesh = plsc.VectorSubcoreMesh(core_axis_name="core", subcore_axis_name="subcore")

    @functools.partial(
        pl.kernel,
        out_type=[jax.ShapeDtypeStruct((32 * acc_len,), F32),
                  jax.ShapeDtypeStruct((32 * dlen,), F32)],
        mesh=mesh,
        scratch_types=[pltpu.VMEM((acc_len,), F32),
                       pltpu.VMEM((dlen,), F32),
                       pltpu.VMEM((gw, ch), F32),
                       pltpu.VMEM((gw, ch), F32),
                       pltpu.VMEM((gw, ch), F32),
                       pltpu.VMEM((gw, ch), F32),
                       pltpu.VMEM((ch,), jnp.int32),
                       pltpu.VMEM((ch,), jnp.int32),
                       pltpu.VMEM((ch,), jnp.int32),
                       pltpu.VMEM((ch,), jnp.int32),
                       pltpu.VMEM((eps,), jnp.int32),
                       pltpu.SemaphoreType.DMA,
                       pltpu.SemaphoreType.DMA,
                       pltpu.SemaphoreType.DMA,
                       pltpu.SemaphoreType.DMA,
                       pltpu.SemaphoreType.DMA,
                       pltpu.SemaphoreType.DMA,
                       pltpu.SemaphoreType.DMA,
                       pltpu.SemaphoreType.DMA],
        compiler_params=_sc_compiler_params(),
    )
    def scatter_kernel(msgt_hbm, dst_hbm, agg_hbm, deg_hbm,
                       acc1, dhist, vals0, vals1, vals2, vals3,
                       raw0, raw1, raw2, raw3, rawd,
                       sg0, sg1, sg2, sg3, sr0, sr1, sr2, sr3):
        t = lax.axis_index("subcore") * 2 + lax.axis_index("core")
        iota = lax.iota(jnp.int32, 16)
        zero16 = jnp.zeros((16,), F32)
        ones16 = jnp.ones((16,), F32)

        valsb = (vals0, vals1, vals2, vals3)
        rawb = (raw0, raw1, raw2, raw3)
        semgb = (sg0, sg1, sg2, sg3)
        semrb = (sr0, sr1, sr2, sr3)

        @pl.loop(0, acc_len, step=16)
        def _(i):
            acc1[pl.ds(i, 16)] = zero16

        @pl.loop(0, dlen, step=16)
        def _(i):
            dhist[pl.ds(i, 16)] = zero16

        def issue(mm, b):
            e0 = mm * ch
            pltpu.async_copy(msgt_hbm.at[pl.ds(t * gw, gw), pl.ds(e0, ch)],
                             valsb[b], semgb[b])
            pltpu.async_copy(dst_hbm.at[pl.ds(e0, ch)], rawb[b], semrb[b])

        def wait(mm, b):
            e0 = mm * ch
            pltpu.make_async_copy(msgt_hbm.at[pl.ds(t * gw, gw), pl.ds(e0, ch)],
                                  valsb[b], semgb[b]).wait()
            pltpu.make_async_copy(dst_hbm.at[pl.ds(e0, ch)], rawb[b], semrb[b]).wait()

        def compute(b):
            @pl.loop(0, ch, step=16)
            def _(j):
                offs0 = rawb[b][pl.ds(j, 16)] * gw
                for r in range(gw):
                    vv = valsb[b][r, pl.ds(j, 16)]
                    plsc.addupdate_scatter(acc1, [offs0 + r], vv)

        issue(0, 0)
        issue(1, 1)
        issue(2, 2)

        main = (nch // 4) * 4 - 4

        @pl.loop(0, main + 4, step=4)
        def _(m):
            for db in range(4):
                mm = m + db

                @pl.when(mm + 3 < nch)
                def _():
                    issue(mm + 3, (db + 3) % 4)

                wait(mm, db)
                compute(db)

        for mm in range(main + 4, nch):
            wait(mm, mm % 4)
            compute(mm % 4)

        e0d = t * eps
        pltpu.sync_copy(dst_hbm.at[pl.ds(e0d, eps)], rawd)

        full16 = (eps // 16) * 16

        @pl.loop(0, full16, step=16)
        def _(j):
            plsc.addupdate_scatter(dhist, [rawd[pl.ds(j, 16)]], ones16)

        if eps != full16:
            tail = rawd[pl.ds(eps - 16, 16)]
            plsc.addupdate_scatter(dhist, [tail], ones16,
                                   mask=iota >= (16 - (eps - full16)))

        pltpu.sync_copy(acc1, agg_hbm.at[pl.ds(t * acc_len, acc_len)])
        pltpu.sync_copy(dhist, deg_hbm.at[pl.ds(t * dlen, dlen)])

    agg_f, deg_f = scatter_kernel(msgt, dst1d)
    agg = agg_f.reshape(32, npad, gw).transpose(1, 0, 2).reshape(npad, h)[:n_nodes]
    deg_t = deg_f.reshape(32, dlen)[:, :n_nodes].T
    return agg, deg_t


def _node_body(nf, pn, agg, deg, w1d, nb1, mb2r, nw2, nb2, lg, lb, out):
    degree = jnp.sum(deg[...], axis=1, keepdims=True)
    aggn = (agg[...] + degree * mb2r[...]) / (degree + 1e-8)
    pre = pn[...] + jnp.dot(aggn, w1d[...], preferred_element_type=F32) + nb1[...]
    hh = _gelu(pre)
    upd = jnp.dot(hh, nw2[...], preferred_element_type=F32) + nb2[...]
    out[...] = _ln_rows(nf[...] + upd, lg[...], lb[...])


def _node_update(nf, pn, agg, deg, w1d, nb1, mb2r, nw2, nb2, lg, lb, blk):
    n, h = nf.shape
    grid = n // blk
    full = lambda a: pl.BlockSpec(a.shape, lambda i: (0, 0))
    rowblk = pl.BlockSpec((blk, h), lambda i: (i, 0))
    return pl.pallas_call(
        _node_body,
        grid=(grid,),
        in_specs=[rowblk, rowblk, rowblk,
                  pl.BlockSpec((blk, 32), lambda i: (i, 0)),
                  full(w1d), full(nb1), full(mb2r), full(nw2), full(nb2),
                  full(lg), full(lb)],
        out_specs=rowblk,
        out_shape=jax.ShapeDtypeStruct((n, h), F32),
    )(nf, pn, agg, deg, w1d, nb1, mb2r, nw2, nb2, lg, lb)


def kernel(node_features, edge_features, edge_index, params):
    p = params
    n, h = node_features.shape
    e = edge_features.shape[0]

    w_all = jnp.concatenate(
        [p['mW1'][0:h], p['eW1'][0:h],
         p['mW1'][h:2 * h], p['eW1'][h:2 * h],
         p['nW1'][0:h]], axis=1)
    b_all = jnp.concatenate(
        [p['mb1'], p['eb1'], jnp.zeros((3 * h,), F32)]).reshape(1, 5 * h)

    tsrc, tdst, pn = _node_proj(node_features, w_all, b_all, blk=1000)

    src = edge_index[0]
    dst = edge_index[1]
    gsrc, gdst = _edge_gather(tsrc, tdst, src, dst)

    msgt, new_edge = _edge_mlp(
        gsrc, gdst, edge_features,
        p['mW1'][2 * h:3 * h], p['eW1'][2 * h:3 * h],
        p['mW2'], p['eW2'], p['eb2'].reshape(1, h),
        p['edge_ln_g'].reshape(1, h), p['edge_ln_b'].reshape(1, h), blk=1280)

    agg, deg = _scatter_agg(msgt, dst, n)

    new_node = _node_update(
        node_features, pn, agg, deg,
        p['nW1'][h:2 * h], p['nb1'].reshape(1, h), p['mb2'].reshape(1, h),
        p['nW2'], p['nb2'].reshape(1, h),
        p['node_ln_g'].reshape(1, h), p['node_ln_b'].reshape(1, h), blk=1000)

    return (new_node, new_edge)
